# baseline (speedup 1.0000x reference)
import jax
import jax.numpy as jnp
from jax import lax
from jax.experimental import pallas as pl

_F32 = jnp.float32


def _row_block(n):
    for br in (200, 80, 40, 8):
        if n % br == 0:
            return br
    return n


def _full(shape):
    return pl.BlockSpec(shape, lambda i: (0,) * len(shape))


def _rows(br, ncols):
    return pl.BlockSpec((br, ncols), lambda i: (i, 0))


def _pre_body(x_ref, w1s_ref, w1f_ref, os_ref, of_ref):
    xb = x_ref[...]
    os_ref[...] = jnp.dot(xb, w1s_ref[...], preferred_element_type=_F32)
    of_ref[...] = jnp.dot(xb, w1f_ref[...], preferred_element_type=_F32)


def _passA_body(as_ref, af_ref, us_ref, uf_ref, b1s_ref, b1f_ref,
                w2s_ref, w2f_ref, ts_ref, tf_ref):
    hs = jnp.maximum(
        jnp.dot(as_ref[...], us_ref[...], preferred_element_type=_F32)
        + b1s_ref[...], 0.0)
    ts_ref[...] = jnp.dot(hs, w2s_ref[...], preferred_element_type=_F32)
    hf = jnp.maximum(
        jnp.dot(af_ref[...], uf_ref[...], preferred_element_type=_F32)
        + b1f_ref[...], 0.0)
    tf_ref[...] = jnp.dot(hf, w2f_ref[...], preferred_element_type=_F32)


def _passB_body(as_ref, af_ref, ts_ref, tf_ref, b2s_ref, b2f_ref,
                attW_ref, attb_ref, attq_ref, mlpW_ref, mlpb_ref,
                decW1_ref, hm_ref, att_ref, hn_ref, hd_ref):
    g_s = (jnp.dot(as_ref[...], ts_ref[...], preferred_element_type=_F32)
           + b2s_ref[...])
    g_f = (jnp.dot(af_ref[...], tf_ref[...], preferred_element_type=_F32)
           + b2f_ref[...])
    w_s = jnp.tanh(jnp.dot(g_s, attW_ref[...], preferred_element_type=_F32)
                   + attb_ref[...])
    w_f = jnp.tanh(jnp.dot(g_f, attW_ref[...], preferred_element_type=_F32)
                   + attb_ref[...])
    sc_s = jnp.dot(w_s, attq_ref[...], preferred_element_type=_F32)
    sc_f = jnp.dot(w_f, attq_ref[...], preferred_element_type=_F32)
    m = jnp.maximum(sc_s, sc_f)
    es = jnp.exp(sc_s - m)
    ef = jnp.exp(sc_f - m)
    den = es + ef
    a_s = es / den
    a_f = ef / den
    h = a_s * g_s + a_f * g_f
    hm = jnp.dot(h, mlpW_ref[...], preferred_element_type=_F32) + mlpb_ref[...]
    hm_ref[...] = hm
    att_ref[...] = jnp.concatenate([a_s, a_f], axis=1)
    nrm = jnp.sqrt(jnp.sum(hm * hm, axis=1, keepdims=True))
    hn_ref[...] = hm / (nrm + 1e-8)
    hd_ref[...] = jnp.dot(hm, decW1_ref[...], preferred_element_type=_F32)


def _passC_body(as_ref, hd_ref, hnb_ref, hn_ref, db1_ref, wpi_ref, bpi_ref,
                wdisp_ref, bdisp_ref, wmean_ref, bmean_ref,
                pi_ref, disp_ref, mean_ref, recon_ref):
    h1 = jnp.maximum(
        jnp.dot(as_ref[...], hd_ref[...], preferred_element_type=_F32)
        + db1_ref[...], 0.0)
    zpi = jnp.dot(h1, wpi_ref[...], preferred_element_type=_F32) + bpi_ref[...]
    pi_ref[...] = 1.0 / (1.0 + jnp.exp(-zpi))
    zd = jnp.dot(h1, wdisp_ref[...], preferred_element_type=_F32) + bdisp_ref[...]
    sp = jnp.maximum(zd, 0.0) + jnp.log1p(jnp.exp(-jnp.abs(zd)))
    disp_ref[...] = jnp.clip(sp, 1e-4, 1e4)
    zm = jnp.dot(h1, wmean_ref[...], preferred_element_type=_F32) + bmean_ref[...]
    mean_ref[...] = jnp.clip(jnp.exp(zm), 1e-5, 1e6)
    recon_ref[...] = lax.dot_general(
        hnb_ref[...], hn_ref[...], (((1,), (1,)), ((), ())),
        preferred_element_type=_F32)


def kernel(x, adj_s, adj_f, params):
    p = params
    n, nfeat = x.shape
    nh1 = p['s_W1'].shape[1]
    nh2 = p['s_W2'].shape[1]
    br = _row_block(n)
    grid = (n // br,)

    def vec2(v):
        return v.reshape(1, -1)

    xw1_s, xw1_f = pl.pallas_call(
        _pre_body,
        grid=grid,
        in_specs=[_rows(br, nfeat), _full((nfeat, nh1)), _full((nfeat, nh1))],
        out_specs=[_rows(br, nh1), _rows(br, nh1)],
        out_shape=[jax.ShapeDtypeStruct((n, nh1), _F32)] * 2,
    )(x, p['s_W1'], p['f_W1'])

    t_s, t_f = pl.pallas_call(
        _passA_body,
        grid=grid,
        in_specs=[_rows(br, n), _rows(br, n),
                  _full((n, nh1)), _full((n, nh1)),
                  _full((1, nh1)), _full((1, nh1)),
                  _full((nh1, nh2)), _full((nh1, nh2))],
        out_specs=[_rows(br, nh2), _rows(br, nh2)],
        out_shape=[jax.ShapeDtypeStruct((n, nh2), _F32)] * 2,
    )(adj_s, adj_f, xw1_s, xw1_f, vec2(p['s_b1']), vec2(p['f_b1']),
      p['s_W2'], p['f_W2'])

    hm, att2, hn, hd = pl.pallas_call(
        _passB_body,
        grid=grid,
        in_specs=[_rows(br, n), _rows(br, n),
                  _full((n, nh2)), _full((n, nh2)),
                  _full((1, nh2)), _full((1, nh2)),
                  _full((nh2, nh2)), _full((1, nh2)), _full((nh2, 1)),
                  _full((nh2, nh2)), _full((1, nh2)),
                  _full((nh2, nh1))],
        out_specs=[_rows(br, nh2), _rows(br, 2), _rows(br, nh2),
                   _rows(br, nh1)],
        out_shape=[jax.ShapeDtypeStruct((n, nh2), _F32),
                   jax.ShapeDtypeStruct((n, 2), _F32),
                   jax.ShapeDtypeStruct((n, nh2), _F32),
                   jax.ShapeDtypeStruct((n, nh1), _F32)],
    )(adj_s, adj_f, t_s, t_f, vec2(p['s_b2']), vec2(p['f_b2']),
      p['att_W'], vec2(p['att_b']), p['att_q'], p['mlp_W'], vec2(p['mlp_b']),
      p['dec_W1'])

    pi, disp, mean, recon = pl.pallas_call(
        _passC_body,
        grid=grid,
        in_specs=[_rows(br, n), _full((n, nh1)),
                  _rows(br, nh2), _full((n, nh2)),
                  _full((1, nh1)),
                  _full((nh1, nfeat)), _full((1, nfeat)),
                  _full((nh1, nfeat)), _full((1, nfeat)),
                  _full((nh1, nfeat)), _full((1, nfeat))],
        out_specs=[_rows(br, nfeat), _rows(br, nfeat), _rows(br, nfeat),
                   _rows(br, n)],
        out_shape=[jax.ShapeDtypeStruct((n, nfeat), _F32),
                   jax.ShapeDtypeStruct((n, nfeat), _F32),
                   jax.ShapeDtypeStruct((n, nfeat), _F32),
                   jax.ShapeDtypeStruct((n, n), _F32)],
    )(adj_s, hd, hn, hn, vec2(p['dec_b1']),
      p['dec_Wpi'], vec2(p['dec_bpi']),
      p['dec_Wdisp'], vec2(p['dec_bdisp']),
      p['dec_Wmean'], vec2(p['dec_bmean']))

    return (hm, recon, pi, disp, mean, att2.reshape(n, 2, 1))
